# initial kernel scaffold (unmeasured)
import jax
import jax.numpy as jnp
from jax import lax
from jax.experimental import pallas as pl
from jax.experimental.pallas import tpu as pltpu


def kernel(
    x,
):
    def body(*refs):
        pass

    out_shape = jax.ShapeDtypeStruct(..., jnp.float32)
    return pl.pallas_call(body, out_shape=out_shape)(...)


# baseline (device time: 1098601 ns/iter reference)
import jax
import jax.numpy as jnp
from jax import lax
from jax.experimental import pallas as pl
from jax.experimental.pallas import tpu as pltpu

K = 8


def kernel(x):
    m_shard, n = x.shape
    half = m_shard // 2
    chunk = half // K
    m_global = 2 * m_shard

    xb = x.astype(jnp.bfloat16)

    def body(x_ref, out_ref, local_sem,
             x_send_sems, x_recv_sems, y_send_sems, y_recv_sems):
        my_x = lax.axis_index("x")
        my_y = lax.axis_index("y")
        other = 1 - my_x
        x_nbr = (other, my_y)
        y_nbr = (my_x, 1 - my_y)

        barrier = pltpu.get_barrier_semaphore()
        for nbr in (x_nbr, y_nbr):
            pl.semaphore_signal(
                barrier, inc=1,
                device_id=nbr, device_id_type=pl.DeviceIdType.MESH,
            )
        pl.semaphore_wait(barrier, 2)

        local = pltpu.make_async_copy(
            x_ref, out_ref.at[pl.ds(my_x * m_shard, m_shard)], local_sem
        )
        local.start()

        send_base = my_y * half
        dst_base = my_x * m_shard + my_y * half
        recv_base = other * m_shard + my_y * half

        x_rdmas = []
        for k in range(K):
            rdma = pltpu.make_async_remote_copy(
                src_ref=x_ref.at[pl.ds(send_base + k * chunk, chunk)],
                dst_ref=out_ref.at[pl.ds(dst_base + k * chunk, chunk)],
                send_sem=x_send_sems.at[k],
                recv_sem=x_recv_sems.at[k],
                device_id=x_nbr,
                device_id_type=pl.DeviceIdType.MESH,
            )
            rdma.start()
            x_rdmas.append(rdma)

        y_rdmas = []
        for k in range(K):
            x_rdmas[k].wait_recv()
            rdma = pltpu.make_async_remote_copy(
                src_ref=out_ref.at[pl.ds(recv_base + k * chunk, chunk)],
                dst_ref=out_ref.at[pl.ds(recv_base + k * chunk, chunk)],
                send_sem=y_send_sems.at[k],
                recv_sem=y_recv_sems.at[k],
                device_id=y_nbr,
                device_id_type=pl.DeviceIdType.MESH,
            )
            rdma.start()
            y_rdmas.append(rdma)

        for k in range(K):
            x_rdmas[k].wait_send()
            y_rdmas[k].wait_send()
            y_rdmas[k].wait_recv()
        local.wait()

    return pl.pallas_call(
        body,
        out_shape=jax.ShapeDtypeStruct((m_global, n), jnp.bfloat16),
        in_specs=[pl.BlockSpec(memory_space=pltpu.MemorySpace.HBM)],
        out_specs=pl.BlockSpec(memory_space=pltpu.MemorySpace.HBM),
        scratch_shapes=[
            pltpu.SemaphoreType.DMA,
            pltpu.SemaphoreType.DMA((K,)),
            pltpu.SemaphoreType.DMA((K,)),
            pltpu.SemaphoreType.DMA((K,)),
            pltpu.SemaphoreType.DMA((K,)),
        ],
        compiler_params=pltpu.CompilerParams(collective_id=0),
    )(xb)


# device time: 295313 ns/iter; 3.7201x vs baseline; 3.7201x over previous
import os

import jax
import jax.numpy as jnp
from jax import lax
from jax.experimental import pallas as pl
from jax.experimental.pallas import tpu as pltpu

K = int(os.environ.get("AG_K", "8"))
_DO_LOCAL = os.environ.get("AG_LOCAL", "1") == "1"
_DO_P1 = os.environ.get("AG_P1", "1") == "1"
_DO_P2 = os.environ.get("AG_P2", "1") == "1"
_VDST = os.environ.get("AG_VDST", "0") == "1"


def kernel(x):
    m_shard, n = x.shape
    half = m_shard // 2
    chunk = half // K
    m_global = 2 * m_shard

    xb = x.astype(jnp.bfloat16)

    def body(x_ref, out_ref, vmem_buf, xrecv_buf, load_sem, store_sem,
             x_send_sems, x_recv_sems, y_send_sems, y_recv_sems):
        my_x = lax.axis_index("x")
        my_y = lax.axis_index("y")
        other = 1 - my_x
        x_nbr = (other, my_y)
        y_nbr = (my_x, 1 - my_y)

        barrier = pltpu.get_barrier_semaphore()
        for nbr in (x_nbr, y_nbr):
            pl.semaphore_signal(
                barrier, inc=1,
                device_id=nbr, device_id_type=pl.DeviceIdType.MESH,
            )
        pl.semaphore_wait(barrier, 2)

        load = pltpu.make_async_copy(x_ref, vmem_buf, load_sem)
        load.start()
        load.wait()

        store = pltpu.make_async_copy(
            vmem_buf, out_ref.at[pl.ds(my_x * m_shard, m_shard)], store_sem
        )
        if _DO_LOCAL:
            store.start()

        send_base = my_y * half
        dst_base = my_x * m_shard + my_y * half
        recv_base = other * m_shard + my_y * half

        x_rdmas = []
        if _DO_P1:
            for k in range(K):
                rdma = pltpu.make_async_remote_copy(
                    src_ref=vmem_buf.at[pl.ds(send_base + k * chunk, chunk)],
                    dst_ref=(
                        xrecv_buf.at[pl.ds(k * chunk, chunk)]
                        if _VDST
                        else out_ref.at[pl.ds(dst_base + k * chunk, chunk)]
                    ),
                    send_sem=x_send_sems.at[k],
                    recv_sem=x_recv_sems.at[k],
                    device_id=x_nbr,
                    device_id_type=pl.DeviceIdType.MESH,
                )
                rdma.start()
                x_rdmas.append(rdma)

        y_rdmas = []
        if _DO_P2:
            for k in range(K):
                if _DO_P1:
                    x_rdmas[k].wait_recv()
                rdma = pltpu.make_async_remote_copy(
                    src_ref=out_ref.at[pl.ds(recv_base + k * chunk, chunk)],
                    dst_ref=out_ref.at[pl.ds(recv_base + k * chunk, chunk)],
                    send_sem=y_send_sems.at[k],
                    recv_sem=y_recv_sems.at[k],
                    device_id=y_nbr,
                    device_id_type=pl.DeviceIdType.MESH,
                )
                rdma.start()
                y_rdmas.append(rdma)

        for k in range(K):
            if _DO_P1:
                x_rdmas[k].wait_send()
                if not _DO_P2:
                    x_rdmas[k].wait_recv()
            if _DO_P2:
                y_rdmas[k].wait_send()
                y_rdmas[k].wait_recv()
        if _DO_LOCAL:
            store.wait()

    return pl.pallas_call(
        body,
        out_shape=jax.ShapeDtypeStruct((m_global, n), jnp.bfloat16),
        in_specs=[pl.BlockSpec(memory_space=pltpu.MemorySpace.HBM)],
        out_specs=pl.BlockSpec(memory_space=pltpu.MemorySpace.HBM),
        scratch_shapes=[
            pltpu.VMEM((m_shard, n), jnp.bfloat16),
            pltpu.VMEM((half, n), jnp.bfloat16),
            pltpu.SemaphoreType.DMA,
            pltpu.SemaphoreType.DMA,
            pltpu.SemaphoreType.DMA((K,)),
            pltpu.SemaphoreType.DMA((K,)),
            pltpu.SemaphoreType.DMA((K,)),
            pltpu.SemaphoreType.DMA((K,)),
        ],
        compiler_params=pltpu.CompilerParams(
            collective_id=0,
            vmem_limit_bytes=64 * 1024 * 1024,
        ),
    )(xb)


# device time: 254345 ns/iter; 4.3193x vs baseline; 1.1611x over previous
import jax
import jax.numpy as jnp
from jax import lax
from jax.experimental import pallas as pl
from jax.experimental.pallas import tpu as pltpu

K = 8


def kernel(x):
    m_shard, n = x.shape
    half = m_shard // 2
    chunk = half // K
    m_global = 2 * m_shard

    def body(x_ref, out_ref, shard_bf16, f32_buf, load_sems, store_sem,
             x_send_sems, x_recv_sems, y_send_sems, y_recv_sems):
        my_x = lax.axis_index("x")
        my_y = lax.axis_index("y")
        other = 1 - my_x
        x_nbr = (other, my_y)
        y_nbr = (my_x, 1 - my_y)

        barrier = pltpu.get_barrier_semaphore()
        for nbr in (x_nbr, y_nbr):
            pl.semaphore_signal(
                barrier, inc=1,
                device_id=nbr, device_id_type=pl.DeviceIdType.MESH,
            )
        pl.semaphore_wait(barrier, 2)

        send_base = my_y * half
        keep_base = (1 - my_y) * half
        dst_base = my_x * m_shard + my_y * half
        recv_base = other * m_shard + my_y * half

        def load_f32(j, slot, row0):
            cp = pltpu.make_async_copy(
                x_ref.at[pl.ds(row0, chunk)], f32_buf.at[slot],
                load_sems.at[slot],
            )
            cp.start()
            return cp

        x_rdmas = []
        pending = load_f32(0, 0, send_base)
        for k in range(K):
            nxt = None
            if k + 1 < K:
                nxt = load_f32(k + 1, (k + 1) % 2, send_base + (k + 1) * chunk)
            pending.wait()
            shard_bf16[pl.ds(send_base + k * chunk, chunk), :] = (
                f32_buf[k % 2, :, :].astype(jnp.bfloat16)
            )
            rdma = pltpu.make_async_remote_copy(
                src_ref=shard_bf16.at[pl.ds(send_base + k * chunk, chunk)],
                dst_ref=out_ref.at[pl.ds(dst_base + k * chunk, chunk)],
                send_sem=x_send_sems.at[k],
                recv_sem=x_recv_sems.at[k],
                device_id=x_nbr,
                device_id_type=pl.DeviceIdType.MESH,
            )
            rdma.start()
            x_rdmas.append(rdma)
            pending = nxt

        pending = load_f32(0, 0, keep_base)
        for k in range(K):
            nxt = None
            if k + 1 < K:
                nxt = load_f32(k + 1, (k + 1) % 2, keep_base + (k + 1) * chunk)
            pending.wait()
            shard_bf16[pl.ds(keep_base + k * chunk, chunk), :] = (
                f32_buf[k % 2, :, :].astype(jnp.bfloat16)
            )
            pending = nxt

        store = pltpu.make_async_copy(
            shard_bf16, out_ref.at[pl.ds(my_x * m_shard, m_shard)], store_sem
        )
        store.start()

        y_rdmas = []
        for k in range(K):
            x_rdmas[k].wait_recv()
            rdma = pltpu.make_async_remote_copy(
                src_ref=out_ref.at[pl.ds(recv_base + k * chunk, chunk)],
                dst_ref=out_ref.at[pl.ds(recv_base + k * chunk, chunk)],
                send_sem=y_send_sems.at[k],
                recv_sem=y_recv_sems.at[k],
                device_id=y_nbr,
                device_id_type=pl.DeviceIdType.MESH,
            )
            rdma.start()
            y_rdmas.append(rdma)

        for k in range(K):
            x_rdmas[k].wait_send()
            y_rdmas[k].wait_send()
            y_rdmas[k].wait_recv()
        store.wait()

    return pl.pallas_call(
        body,
        out_shape=jax.ShapeDtypeStruct((m_global, n), jnp.bfloat16),
        in_specs=[pl.BlockSpec(memory_space=pltpu.MemorySpace.HBM)],
        out_specs=pl.BlockSpec(memory_space=pltpu.MemorySpace.HBM),
        scratch_shapes=[
            pltpu.VMEM((m_shard, n), jnp.bfloat16),
            pltpu.VMEM((2, chunk, n), jnp.float32),
            pltpu.SemaphoreType.DMA((2,)),
            pltpu.SemaphoreType.DMA,
            pltpu.SemaphoreType.DMA((K,)),
            pltpu.SemaphoreType.DMA((K,)),
            pltpu.SemaphoreType.DMA((K,)),
            pltpu.SemaphoreType.DMA((K,)),
        ],
        compiler_params=pltpu.CompilerParams(
            collective_id=0,
            vmem_limit_bytes=64 * 1024 * 1024,
        ),
    )(x)
